# Initial kernel scaffold; baseline (speedup 1.0000x reference)
import jax
import jax.numpy as jnp
from jax.experimental import pallas as pl


def kernel(feat, label, wei, centers):
    raise NotImplementedError("write your pallas kernel here")


# speedup vs baseline: 4.6159x; 4.6159x over previous
import functools

import jax
import jax.numpy as jnp
from jax import lax
from jax.experimental import pallas as pl
from jax.experimental.pallas import tpu as pltpu
from jax.experimental.pallas import tpu_sc as plsc

_FEAT = 128
_BATCH = 16384
_NC = 2
_NS = 16
_NW = _NC * _NS
_ROWS = _BATCH // _NW
_L = 16
_CH = _FEAT // _L


def _sc_body(feat_hbm, label_hbm, wei_hbm, centers_hbm, out_hbm,
             fbuf, lbuf, wbuf, cbuf, obuf):
    wid = lax.axis_index("s") * _NC + lax.axis_index("c")
    base = wid * _ROWS
    pltpu.sync_copy(feat_hbm.at[pl.ds(base, _ROWS), :], fbuf)
    pltpu.sync_copy(label_hbm.at[pl.ds(base, _ROWS)], lbuf)
    pltpu.sync_copy(wei_hbm.at[pl.ds(base, _ROWS)], wbuf)
    pltpu.sync_copy(centers_hbm, cbuf)

    zero = jnp.zeros((_L,), jnp.float32)
    init = (
        tuple(zero for _ in range(_CH)),
        tuple(zero for _ in range(_CH)),
        tuple(zero for _ in range(_CH)),
        tuple(zero for _ in range(_CH)),
        zero, zero, zero,
    )

    def group_step(g, carry):
        qv, pt, av, bv, wv_s, wav_s, wbv_s = carry
        r0 = g * _L
        w16 = wbuf[pl.ds(r0, _L)]
        lf16 = lbuf[pl.ds(r0, _L)].astype(jnp.float32)
        a16 = w16 * lf16
        b16 = a16 * lf16
        qv = list(qv)
        pt = list(pt)
        av = list(av)
        bv = list(bv)
        for j in range(_L):
            w = w16[j]
            a = a16[j]
            b = b16[j]
            for d in range(_CH):
                f = fbuf[r0 + j, pl.ds(d * _L, _L)]
                wf = w * f
                qv[d] = qv[d] + wf * f
                pt[d] = pt[d] + wf
                av[d] = av[d] + a * f
                bv[d] = bv[d] + b * f
        return (tuple(qv), tuple(pt), tuple(av), tuple(bv),
                wv_s + w16, wav_s + a16, wbv_s + b16)

    qv, pt, av, bv, wv_s, wav_s, wbv_s = lax.fori_loop(
        0, _ROWS // _L, group_step, init)
    w_s = jnp.sum(wv_s)
    wa_s = jnp.sum(wav_s)
    wb_s = jnp.sum(wbv_s)

    q16 = zero
    s1_16 = zero
    s2_16 = zero
    for d in range(_CH):
        c0 = cbuf[0, pl.ds(d * _L, _L)]
        c1 = cbuf[1, pl.ds(d * _L, _L)]
        c2 = cbuf[2, pl.ds(d * _L, _L)]
        u = c0
        v = 0.5 * (-3.0 * c0 + 4.0 * c1 - c2)
        z = 0.5 * (c0 - 2.0 * c1 + c2)
        q16 = q16 + qv[d]
        s1_16 = s1_16 + pt[d] * u + av[d] * v + bv[d] * z
        s2_16 = s2_16 + pt[d] * (c0 + c1 + c2)

    q_sc = jnp.sum(q16)
    s1_sc = jnp.sum(s1_16)
    s2_sc = jnp.sum(s2_16)

    lanes = lax.iota(jnp.int32, 16)
    outv = jnp.where(lanes == 0, q_sc, 0.0)
    outv = jnp.where(lanes == 1, s1_sc, outv)
    outv = jnp.where(lanes == 2, s2_sc, outv)
    outv = jnp.where(lanes == 3, w_s, outv)
    outv = jnp.where(lanes == 4, wa_s, outv)
    outv = jnp.where(lanes == 5, wb_s, outv)
    obuf[...] = outv
    pltpu.sync_copy(obuf, out_hbm.at[wid])


@jax.jit
def _partials(feat, label, wei, centers):
    mesh = plsc.VectorSubcoreMesh(core_axis_name="c", subcore_axis_name="s")
    return pl.kernel(
        _sc_body,
        out_type=jax.ShapeDtypeStruct((_NW, _L), jnp.float32),
        mesh=mesh,
        compiler_params=pltpu.CompilerParams(needs_layout_passes=False),
        scratch_types=[
            pltpu.VMEM((_ROWS, _FEAT), jnp.float32),
            pltpu.VMEM((_ROWS,), jnp.int32),
            pltpu.VMEM((_ROWS,), jnp.float32),
            pltpu.VMEM((3, _FEAT), jnp.float32),
            pltpu.VMEM((_L,), jnp.float32),
        ],
    )(feat, label, wei, centers)


def kernel(feat, label, wei, centers):
    part = _partials(feat, label.astype(jnp.int32), wei, centers)
    t = part.sum(axis=0)
    q, s1, s2, w, wa, wb = t[0], t[1], t[2], t[3], t[4], t[5]
    ck2 = (centers * centers).sum(axis=1)
    t_own = (w * ck2[0]
             + wa * 0.5 * (-3.0 * ck2[0] + 4.0 * ck2[1] - ck2[2])
             + wb * 0.5 * (ck2[0] - 2.0 * ck2[1] + ck2[2]))
    s_own = q - 2.0 * s1 + t_own
    s_all = 3.0 * q - 2.0 * s2 + w * (ck2[0] + ck2[1] + ck2[2])
    distocen = s_all - s_own
    return s_own * (1.0 + 1.0 / distocen) / 2.0 / _BATCH
